# baseline (speedup 1.0000x reference)
import functools

import jax
import jax.numpy as jnp
from jax import lax
from jax.experimental import pallas as pl
from jax.experimental.pallas import tpu as pltpu
from jax.experimental.pallas import tpu_sc as plsc

NUM_SAMPLES = 100000
NUM_FEATURES = 64
NUM_CLUSTERS = 2000
C_PAD = 2048
BATCH = 1024
TEMP = 0.05

NC = 2
NS = 16
NW = NC * NS
CHUNK = 800
NCHUNK = NUM_SAMPLES // CHUNK
SUB = 80
NSUB = CHUNK // SUB
MAXT = (NCHUNK + NW - 1) // NW
QPW = BATCH // NW


def _sc_body(feat_hbm, labflat_hbm, idxflat_hbm, zb_hbm, zc_hbm,
             ones_hbm, bank_out, cnt_out, tgt_out,
             idx_v, rows_v, ones_v, qidx_v, tgt_v, bank_sh, cnt_sh,
             lsem, ssem, gsem):
    cid = lax.axis_index("c")
    sid = lax.axis_index("s")
    wid = sid * NC + cid

    def load(t, b):
        j = wid + NW * t
        r = pltpu.async_copy(feat_hbm.at[pl.ds(j * CHUNK, CHUNK), :],
                             rows_v.at[b], lsem[b])
        i = pltpu.async_copy(labflat_hbm.at[pl.ds(j * CHUNK, CHUNK)],
                             idx_v.at[b], lsem[b])
        return (r, i)

    ld = [None, None]
    ld[0] = load(0, 0)

    @pl.when(sid == 0)
    def _():
        pltpu.sync_copy(zb_hbm, bank_sh)
        pltpu.sync_copy(zc_hbm, cnt_sh)

    pltpu.sync_copy(ones_hbm, ones_v)
    pltpu.sync_copy(idxflat_hbm.at[pl.ds(wid * QPW, QPW)], qidx_v)
    pltpu.async_copy(labflat_hbm.at[qidx_v], tgt_v, gsem).wait()
    pltpu.sync_copy(tgt_v, tgt_out.at[pl.ds(wid * QPW, QPW)])

    plsc.subcore_barrier()

    has_t3 = wid < NCHUNK - 3 * NW

    def fire_scatters(b):
        out = []
        for r in range(NSUB):
            out.append(
                pltpu.async_copy(rows_v.at[b, pl.ds(r * SUB, SUB), :],
                                 bank_sh.at[idx_v.at[b, pl.ds(r * SUB, SUB)]], ssem[b],
                                 add=True))
            out.append(
                pltpu.async_copy(ones_v, cnt_sh.at[idx_v.at[b, pl.ds(r * SUB, SUB)]],
                                 ssem[b], add=True))
        return out

    scat = [[], []]
    for t in range(3):
        b = t & 1
        for d in ld[b]:
            d.wait()
        scat[b] = fire_scatters(b)
        nb = 1 - b
        for d in scat[nb]:
            d.wait()
        scat[nb] = []
        if t < 2:
            ld[nb] = load(t + 1, nb)
        else:
            @pl.when(has_t3)
            def _():
                ld3 = load(3, nb)
                for d in ld3:
                    d.wait()
                for d in fire_scatters(nb):
                    d.wait()

    for d in scat[0]:
        d.wait()

    plsc.subcore_barrier()

    @pl.when(sid == 0)
    def _():
        pltpu.sync_copy(bank_sh, bank_out.at[cid])
        pltpu.sync_copy(cnt_sh, cnt_out.at[cid])


_sc_call = functools.partial(
    pl.kernel,
    out_type=(
        jax.ShapeDtypeStruct((NC, C_PAD, NUM_FEATURES), jnp.float32),
        jax.ShapeDtypeStruct((NC, C_PAD, 16), jnp.float32),
        jax.ShapeDtypeStruct((BATCH,), jnp.int32),
    ),
    mesh=plsc.VectorSubcoreMesh(core_axis_name="c", subcore_axis_name="s"),
    compiler_params=pltpu.CompilerParams(use_tc_tiling_on_sc=False),
    scratch_types=(
        pltpu.VMEM((2, CHUNK), jnp.int32),
        pltpu.VMEM((2, CHUNK, NUM_FEATURES), jnp.float32),
        pltpu.VMEM((SUB, 16), jnp.float32),
        pltpu.VMEM((QPW,), jnp.int32),
        pltpu.VMEM((QPW,), jnp.int32),
        pltpu.VMEM_SHARED((C_PAD, NUM_FEATURES), jnp.float32),
        pltpu.VMEM_SHARED((C_PAD, 16), jnp.float32),
        (pltpu.SemaphoreType.DMA, pltpu.SemaphoreType.DMA),
        (pltpu.SemaphoreType.DMA, pltpu.SemaphoreType.DMA),
        pltpu.SemaphoreType.DMA,
    ),
)(_sc_body)


def _tc_body(x_ref, bank_ref, cnt_ref, tgt_ref, out_ref):
    x = x_ref[...]
    bank = bank_ref[0] + bank_ref[1]
    cnt = cnt_ref[0, :, 0:1] + cnt_ref[1, :, 0:1]
    dots = lax.dot_general(x, bank, (((1,), (1,)), ((), ())),
                           preferred_element_type=jnp.float32,
                           precision=lax.Precision.HIGHEST)
    denom = jnp.where(cnt > 0.0, cnt, 1.0)
    scale = (1.0 / TEMP) / denom
    vec = dots * scale.T
    mask = (cnt > 0.0).astype(jnp.float32).T
    exps = jnp.exp(vec) * mask
    sums = jnp.sum(exps, axis=1, keepdims=True) + 1e-6
    cids = lax.broadcasted_iota(jnp.int32, exps.shape, 1)
    texp = jnp.sum(jnp.where(cids == tgt_ref[...], exps, 0.0),
                   axis=1, keepdims=True)
    logp = jnp.log(texp / sums + 1e-6)
    out_ref[...] = -jnp.sum(logp, axis=0, keepdims=True) / float(BATCH)


_tc_call = pl.pallas_call(
    _tc_body,
    out_shape=jax.ShapeDtypeStruct((1, 1), jnp.float32),
)


def kernel(inputs, indexes, features, label):
    zb = jnp.zeros((C_PAD, NUM_FEATURES), jnp.float32)
    zc = jnp.zeros((C_PAD, 16), jnp.float32)
    ones = jnp.ones((SUB, 16), jnp.float32)
    bank2, cnt2, tgt = _sc_call(features, label, indexes, zb, zc, ones)
    loss = _tc_call(inputs, bank2, cnt2, tgt.reshape(BATCH, 1))
    return loss.reshape(())
